# Initial kernel scaffold; baseline (speedup 1.0000x reference)
import jax
import jax.numpy as jnp
from jax.experimental import pallas as pl


def kernel(x1, x2, edge_index1, edge_index2, adj_vals1, adj_vals2, Wa1, Wa2, Wg1, Wb1, Wb2, Wg2, Wd, bd, D_W1, D_b1, D_W2, D_b2):
    raise NotImplementedError("write your pallas kernel here")


# speedup vs baseline: 1.4154x; 1.4154x over previous
import functools

import jax
import jax.numpy as jnp
from jax import lax
from jax.experimental import pallas as pl
from jax.experimental.pallas import tpu as pltpu
from jax.experimental.pallas import tpu_sc as plsc

N = 4096
E = 131072
RPT = N // 16
CAP = 8960
PADB = 64
SCN = 1024
CHK = 64

_mesh = plsc.VectorSubcoreMesh(core_axis_name="c", subcore_axis_name="s")


def _partition_pair(src1, dst1, val1, src2, dst2, val2):
    @functools.partial(
        pl.kernel,
        out_type=(jax.ShapeDtypeStruct((16, CAP), jnp.int32),
                  jax.ShapeDtypeStruct((16, CAP), jnp.int32),
                  jax.ShapeDtypeStruct((16, CAP), jnp.float32),
                  jax.ShapeDtypeStruct((16, CAP), jnp.int32),
                  jax.ShapeDtypeStruct((16, CAP), jnp.int32),
                  jax.ShapeDtypeStruct((16, CAP), jnp.float32)),
        mesh=_mesh,
        compiler_params=pltpu.CompilerParams(needs_layout_passes=False),
        scratch_types=[
            pltpu.VMEM((SCN,), jnp.int32),
            pltpu.VMEM((SCN,), jnp.int32),
            pltpu.VMEM((SCN,), jnp.float32),
            pltpu.VMEM((CAP + PADB,), jnp.int32),
            pltpu.VMEM((CAP + PADB,), jnp.int32),
            pltpu.VMEM((CAP + PADB,), jnp.float32),
        ],
    )
    def k(s1h, d1h, v1h, s2h, d2h, v2h, os1, od1, ov1, os2, od2, ov2,
          sbuf, dbuf, vbuf, ls, ld, lv):
        c = lax.axis_index("c")
        s = lax.axis_index("s")
        lo = s * RPT

        pad_src = (jnp.arange(16, dtype=jnp.int32) * 257) % N
        zero16i = jnp.zeros((16,), jnp.int32)
        zero16f = jnp.zeros((16,), jnp.float32)

        def prefill(i, _):
            sl = pl.ds(i * 16, 16)
            ls[sl] = pad_src
            ld[sl] = zero16i
            lv[sl] = zero16f
            return 0

        lax.fori_loop(0, (CAP + PADB) // 16, prefill, 0)

        def run(sh, dh, vh):
            def chunk(i, off):
                base = i * SCN
                pltpu.sync_copy(sh.at[pl.ds(base, SCN)], sbuf)
                pltpu.sync_copy(dh.at[pl.ds(base, SCN)], dbuf)
                pltpu.sync_copy(vh.at[pl.ds(base, SCN)], vbuf)

                def grp(j, off):
                    sl = pl.ds(j * 16, 16)
                    dvec = dbuf[sl]
                    m = (dvec >= lo) & (dvec < lo + RPT)
                    pos = plsc.cumsum(m.astype(jnp.int32)) + (off - 1)
                    plsc.store_scatter(ls, [pos], sbuf[sl], mask=m)
                    plsc.store_scatter(ld, [pos], dvec - lo, mask=m)
                    plsc.store_scatter(lv, [pos], vbuf[sl], mask=m)
                    return pos[15] + 1

                return lax.fori_loop(0, SCN // 16, grp, off)

            lax.fori_loop(0, E // SCN, chunk, jnp.int32(0))

        @pl.when(c == 0)
        def _():
            run(s1h, d1h, v1h)
            pltpu.sync_copy(ls.at[pl.ds(0, CAP)], os1.at[s])
            pltpu.sync_copy(ld.at[pl.ds(0, CAP)], od1.at[s])
            pltpu.sync_copy(lv.at[pl.ds(0, CAP)], ov1.at[s])

        @pl.when(c == 1)
        def _():
            run(s2h, d2h, v2h)
            pltpu.sync_copy(ls.at[pl.ds(0, CAP)], os2.at[s])
            pltpu.sync_copy(ld.at[pl.ds(0, CAP)], od2.at[s])
            pltpu.sync_copy(lv.at[pl.ds(0, CAP)], ov2.at[s])

    return k(src1, dst1, val1, src2, dst2, val2)


def _spmm_pair(x1, x2, b1, b2, D):
    @functools.partial(
        pl.kernel,
        out_type=(jax.ShapeDtypeStruct((N, D), jnp.float32),
                  jax.ShapeDtypeStruct((N, D), jnp.float32)),
        mesh=_mesh,
        scratch_types=[
            pltpu.VMEM((CAP,), jnp.int32),
            pltpu.VMEM((CAP,), jnp.int32),
            pltpu.VMEM((CAP,), jnp.float32),
            pltpu.VMEM((CHK, D), jnp.float32),
            pltpu.VMEM((RPT, D), jnp.float32),
            pltpu.SemaphoreType.DMA,
        ],
    )
    def k(x1h, x2h, s1h, d1h, v1h, s2h, d2h, v2h, o1h, o2h,
          lsv, ldv, lvv, rows, acc, sem):
        c = lax.axis_index("c")
        s = lax.axis_index("s")

        zero16 = jnp.zeros((16,), jnp.float32)

        def zrow(i, _):
            for q in range(D // 16):
                acc[i, pl.ds(q * 16, 16)] = zero16
            return 0

        lax.fori_loop(0, RPT, zrow, 0)

        def run(xh, sh, dh, vh, oh):
            pltpu.sync_copy(sh.at[s], lsv)
            pltpu.sync_copy(dh.at[s], ldv)
            pltpu.sync_copy(vh.at[s], lvv)

            def chunk(i, _):
                pltpu.async_copy(
                    xh.at[lsv.at[pl.ds(i * CHK, CHK)]], rows, sem).wait()

                def grp(j, _):
                    sl = pl.ds(i * CHK + j * 16, 16)
                    dvec = ldv[sl]
                    vvec = lvv[sl]
                    for l in range(16):
                        d = dvec[l]
                        v = vvec[l]
                        e = j * 16 + l
                        for q in range(D // 16):
                            qs = pl.ds(q * 16, 16)
                            plsc.addupdate(acc.at[d, qs], rows[e, qs] * v)
                    return 0

                lax.fori_loop(0, CHK // 16, grp, 0)
                return 0

            lax.fori_loop(0, CAP // CHK, chunk, 0)
            pltpu.sync_copy(acc, oh.at[pl.ds(s * RPT, RPT), :])

        @pl.when(c == 0)
        def _():
            run(x1h, s1h, d1h, v1h, o1h)

        @pl.when(c == 1)
        def _():
            run(x2h, s2h, d2h, v2h, o2h)

    return k(x1, x2, b1[0], b1[1], b1[2], b2[0], b2[1], b2[2])


def _dg(a, b):
    return lax.dot_general(a, b, (((1,), (0,)), ((), ())),
                           preferred_element_type=jnp.float32,
                           precision=lax.Precision.HIGHEST)


def _dgt(a, b):
    return lax.dot_general(a, b, (((1,), (1,)), ((), ())),
                           preferred_element_type=jnp.float32,
                           precision=lax.Precision.HIGHEST)


def _mm(x, w, bm, relu_in=False):
    m, kdim = x.shape
    _, n = w.shape

    def body(xr, wr, outr):
        xv = xr[...]
        if relu_in:
            xv = jnp.maximum(xv, 0.0)
        outr[...] = _dg(xv, wr[...])

    return pl.pallas_call(
        body,
        grid=(m // bm,),
        in_specs=[pl.BlockSpec((bm, kdim), lambda i: (i, 0)),
                  pl.BlockSpec((kdim, n), lambda i: (0, 0))],
        out_specs=pl.BlockSpec((bm, n), lambda i: (i, 0)),
        out_shape=jax.ShapeDtypeStruct((m, n), jnp.float32),
    )(x, w)


def _decoders(z1, z2, wd, bdr, dw1, db1r, dw2r):
    bm = 256
    h2 = z1.shape[1]
    f = wd.shape[1]
    dh = dw1.shape[1]

    def body(z1r, z1fr, z2r, z2fr, wdr, bdr_, dw1r, db1r_, dw2r_,
             r1r, r2r, atr, fkr):
        i = pl.program_id(0)
        b1 = z1r[...]
        b2 = z2r[...]
        r1r[...] = _dgt(b1, z1fr[...])
        r2r[...] = _dgt(b2, z2fr[...])
        atr[...] = jnp.maximum(_dg(b1, wdr[...]) + bdr_[...], 0.0)
        hd = jnp.maximum(_dg(b2, dw1r[...]) + db1r_[...], 0.0)
        part = jnp.sum(hd * dw2r_[...]).reshape(1, 1)

        @pl.when(i == 0)
        def _():
            fkr[...] = jnp.zeros((1, 1), jnp.float32)

        fkr[...] += part

    c0 = lambda i: (0, 0)
    return pl.pallas_call(
        body,
        grid=(N // bm,),
        in_specs=[
            pl.BlockSpec((bm, h2), lambda i: (i, 0)),
            pl.BlockSpec((N, h2), c0),
            pl.BlockSpec((bm, h2), lambda i: (i, 0)),
            pl.BlockSpec((N, h2), c0),
            pl.BlockSpec((h2, f), c0),
            pl.BlockSpec((1, f), c0),
            pl.BlockSpec((h2, dh), c0),
            pl.BlockSpec((1, dh), c0),
            pl.BlockSpec((1, dh), c0),
        ],
        out_specs=[
            pl.BlockSpec((bm, N), lambda i: (i, 0)),
            pl.BlockSpec((bm, N), lambda i: (i, 0)),
            pl.BlockSpec((bm, f), lambda i: (i, 0)),
            pl.BlockSpec((1, 1), c0),
        ],
        out_shape=[
            jax.ShapeDtypeStruct((N, N), jnp.float32),
            jax.ShapeDtypeStruct((N, N), jnp.float32),
            jax.ShapeDtypeStruct((N, f), jnp.float32),
            jax.ShapeDtypeStruct((1, 1), jnp.float32),
        ],
    )(z1, z1, z2, z2, wd, bdr, dw1, db1r, dw2r)


def kernel(x1, x2, edge_index1, edge_index2, adj_vals1, adj_vals2,
           Wa1, Wa2, Wg1, Wb1, Wb2, Wg2, Wd, bd, D_W1, D_b1, D_W2, D_b2):
    src1, dst1 = edge_index1[0], edge_index1[1]
    src2, dst2 = edge_index2[0], edge_index2[1]

    bs1, bd1, bv1, bs2, bd2, bv2 = _partition_pair(
        src1, dst1, adj_vals1, src2, dst2, adj_vals2)
    b1 = (bs1, bd1, bv1)
    b2 = (bs2, bd2, bv2)

    xw1 = _mm(x1, Wa1, 512)
    xw2 = _mm(x2, Wa2, 512)
    h1, h2 = _spmm_pair(xw1, xw2, b1, b2, 256)
    t1 = _mm(h1, Wg1, 512, relu_in=True)
    t2 = _mm(h2, Wg1, 512, relu_in=True)
    z1, z2 = _spmm_pair(t1, t2, b1, b2, 128)

    rec1, rec2, attr, fk = _decoders(
        z1, z2, Wd, bd.reshape(1, -1),
        D_W1, D_b1.reshape(1, -1), D_W2.reshape(1, -1))
    fake_prob = fk[0, 0] / jnp.float32(N) + D_b2[0]
    return rec1.reshape(-1), rec2.reshape(-1), attr, fake_prob
